# baseline (speedup 1.0000x reference)
import jax
import jax.numpy as jnp
from jax import lax
from jax.experimental import pallas as pl
from jax.experimental.pallas import tpu as pltpu
from jax.experimental.pallas import tpu_sc as plsc

N_USERS = 50000
N_ITEMS = 50000
N_NODES = N_USERS + N_ITEMS
N_EDGES = 1600000
D = 32
BATCH = 16384

NC = 2
NS = 16
SUB = 128
SUBS = 16
CHUNK = SUB * SUBS
CAP_CHUNKS = (N_EDGES + CHUNK - 1) // CHUNK + 2
CAP = CAP_CHUNKS * CHUNK

HALF = N_NODES // NC
DUMMY = HALF
ACC_ROWS = HALF + 8
STRIPE = 3128
LAST_STRIPE = HALF - (NS - 1) * STRIPE

_mesh = plsc.VectorSubcoreMesh(
    core_axis_name="c", subcore_axis_name="s", num_cores=NC, num_subcores=NS
)


def _layer_body(emb_in, src_hbm, dst_hbm, val_hbm, zeros, meta_hbm, emb_out,
                src_v, dst_v, val_v, rows_v, meta_v, acc_sh, sem):
    c = lax.axis_index("c")
    s = lax.axis_index("s")
    c0 = c * HALF

    @pl.when(s < NS - 1)
    def _zero_full():
        pltpu.sync_copy(zeros, acc_sh.at[pl.ds(s * STRIPE, STRIPE)])

    @pl.when(s == NS - 1)
    def _zero_last():
        pltpu.sync_copy(zeros.at[pl.ds(0, LAST_STRIPE)],
                        acc_sh.at[pl.ds(s * STRIPE, LAST_STRIPE)])

    pltpu.sync_copy(meta_hbm, meta_v)
    mv = meta_v[pl.ds(0, 16)]
    n_chunks = jnp.where(c == 0, mv[0], mv[1])
    base_chunk = jnp.where(c == 0, 0, mv[0])

    plsc.subcore_barrier()

    def one_chunk(k):
        off = (base_chunk + k) * SUBS
        pltpu.async_copy(src_hbm.at[pl.ds(off, SUBS)], src_v, sem).wait()
        pltpu.async_copy(dst_hbm.at[pl.ds(off, SUBS)], dst_v, sem).wait()
        pltpu.async_copy(val_hbm.at[pl.ds(off, SUBS)], val_v, sem).wait()

        for j in range(SUBS):
            for k2 in range(SUB // 16):
                d = dst_v[j, pl.ds(k2 * 16, 16)]
                loc = d - c0
                ok = (d >= c0) & (loc < HALF)
                dst_v[j, pl.ds(k2 * 16, 16)] = jnp.where(ok, loc, DUMMY)

        for j in range(SUBS):
            pltpu.async_copy(emb_in.at[src_v.at[j]], rows_v, sem).wait()

            def scale_body(g, _):
                vv = val_v[j, pl.ds(g * 16, 16)]
                for i2 in range(16):
                    e = g * 16 + i2
                    vi = vv[i2]
                    rows_v[e, pl.ds(0, 16)] = rows_v[e, pl.ds(0, 16)] * vi
                    rows_v[e, pl.ds(16, 16)] = rows_v[e, pl.ds(16, 16)] * vi
                return 0

            lax.fori_loop(0, SUB // 16, scale_body, 0)
            pltpu.async_copy(rows_v, acc_sh.at[dst_v.at[j]], sem,
                             add=True).wait()

    def cond(m):
        return s + m * NS < n_chunks

    def body(m):
        one_chunk(s + m * NS)
        return m + 1

    lax.while_loop(cond, body, jnp.int32(0))

    plsc.subcore_barrier()

    @pl.when(s < NS - 1)
    def _wb_full():
        pltpu.sync_copy(acc_sh.at[pl.ds(s * STRIPE, STRIPE)],
                        emb_out.at[pl.ds(c0 + s * STRIPE, STRIPE)])

    @pl.when(s == NS - 1)
    def _wb_last():
        pltpu.sync_copy(acc_sh.at[pl.ds(s * STRIPE, LAST_STRIPE)],
                        emb_out.at[pl.ds(c0 + s * STRIPE, LAST_STRIPE)])


_params = pltpu.CompilerParams(use_tc_tiling_on_sc=False,
                               needs_layout_passes=False)

_layer = pl.kernel(
    _layer_body,
    out_type=jax.ShapeDtypeStruct((N_NODES, D), jnp.float32),
    mesh=_mesh,
    compiler_params=_params,
    scratch_types=[
        pltpu.VMEM((SUBS, SUB), jnp.int32),
        pltpu.VMEM((SUBS, SUB), jnp.int32),
        pltpu.VMEM((SUBS, SUB), jnp.float32),
        pltpu.VMEM((SUB, D), jnp.float32),
        pltpu.VMEM((16,), jnp.int32),
        pltpu.VMEM_SHARED((ACC_ROWS, D), jnp.float32),
        pltpu.SemaphoreType.DMA,
    ],
)

B_PER_W = BATCH // (NC * NS)
BROWS_PER_W = B_PER_W // SUB


def _final_body(emb0, emb1, emb2, emb3, users2d, pos2d,
                ue, pe, uf, pf, idx_v, acc_v, sem):
    c = lax.axis_index("c")
    s = lax.axis_index("s")
    wid = s * NC + c
    row0 = wid * BROWS_PER_W
    base = wid * B_PER_W

    def lookup(idx2d, offset, out_raw, out_final):
        pltpu.async_copy(idx2d.at[pl.ds(row0, BROWS_PER_W)], idx_v, sem).wait()
        if offset:
            for j in range(BROWS_PER_W):
                for k in range(SUB // 16):
                    idx_v[j, pl.ds(k * 16, 16)] = (
                        idx_v[j, pl.ds(k * 16, 16)] + offset)
        for j in range(BROWS_PER_W):
            pltpu.async_copy(emb0.at[idx_v.at[j]],
                             acc_v.at[pl.ds(j * SUB, SUB)], sem).wait()
        pltpu.sync_copy(acc_v, out_raw.at[pl.ds(base, B_PER_W)])
        for emb in (emb1, emb2, emb3):
            for j in range(BROWS_PER_W):
                pltpu.async_copy(emb.at[idx_v.at[j]],
                                 acc_v.at[pl.ds(j * SUB, SUB)], sem,
                                 add=True).wait()

        def scale_body(i, _):
            acc_v[i, pl.ds(0, 16)] = acc_v[i, pl.ds(0, 16)] * 0.25
            acc_v[i, pl.ds(16, 16)] = acc_v[i, pl.ds(16, 16)] * 0.25
            return 0

        lax.fori_loop(0, B_PER_W, scale_body, 0)
        pltpu.sync_copy(acc_v, out_final.at[pl.ds(base, B_PER_W)])

    lookup(users2d, 0, ue, uf)
    lookup(pos2d, N_USERS, pe, pf)


_final = pl.kernel(
    _final_body,
    out_type=(
        jax.ShapeDtypeStruct((BATCH, D), jnp.float32),
        jax.ShapeDtypeStruct((BATCH, D), jnp.float32),
        jax.ShapeDtypeStruct((BATCH, D), jnp.float32),
        jax.ShapeDtypeStruct((BATCH, D), jnp.float32),
    ),
    mesh=_mesh,
    compiler_params=_params,
    scratch_types=[
        pltpu.VMEM((BROWS_PER_W, SUB), jnp.int32),
        pltpu.VMEM((B_PER_W, D), jnp.float32),
        pltpu.SemaphoreType.DMA,
    ],
)


def kernel(user_table, item_table, edge_val, edge_src, edge_dst, users, pos_items):
    emb0 = jnp.concatenate([user_table, item_table], axis=0)

    edge_src = edge_src.astype(jnp.int32)
    edge_dst = edge_dst.astype(jnp.int32)
    low = edge_dst < HALF
    lrank = jnp.cumsum(low.astype(jnp.int32))
    hrank = jnp.cumsum(1 - low.astype(jnp.int32))
    n_low = lrank[-1]
    cl = (n_low + CHUNK - 1) // CHUNK
    ch = (N_EDGES - n_low + CHUNK - 1) // CHUNK
    pos = jnp.where(low, lrank - 1, cl * CHUNK + hrank - 1)
    src_p = jnp.zeros((CAP,), jnp.int32).at[pos].set(edge_src)
    dst_p = jnp.full((CAP,), N_NODES, jnp.int32).at[pos].set(edge_dst)
    val_p = jnp.zeros((CAP,), jnp.float32).at[pos].set(edge_val)
    src_p = src_p.reshape(CAP // SUB, SUB)
    dst_p = dst_p.reshape(CAP // SUB, SUB)
    val_p = val_p.reshape(CAP // SUB, SUB)
    meta = jnp.zeros((16,), jnp.int32).at[0].set(cl).at[1].set(ch)
    zeros = jnp.zeros((STRIPE, D), jnp.float32)

    e1 = _layer(emb0, src_p, dst_p, val_p, zeros, meta)
    e2 = _layer(e1, src_p, dst_p, val_p, zeros, meta)
    e3 = _layer(e2, src_p, dst_p, val_p, zeros, meta)

    users2d = users.astype(jnp.int32).reshape(BATCH // SUB, SUB)
    pos2d = pos_items.astype(jnp.int32).reshape(BATCH // SUB, SUB)
    return _final(emb0, e1, e2, e3, users2d, pos2d)


# speedup vs baseline: 6.5416x; 6.5416x over previous
import jax
import jax.numpy as jnp
from jax import lax
from jax.experimental import pallas as pl
from jax.experimental.pallas import tpu as pltpu
from jax.experimental.pallas import tpu_sc as plsc

N_USERS = 50000
N_ITEMS = 50000
N_NODES = N_USERS + N_ITEMS
N_EDGES = 1600000
D = 32
HD = 16
BATCH = 16384

NC = 2
NS = 16
SUB = 128
SUBS = 16
CHUNK = SUB * SUBS
E_PAD = ((N_EDGES + NS * CHUNK - 1) // (NS * CHUNK)) * (NS * CHUNK)
ROWS = E_PAD // SUB
ROWS_PER_TILE = ROWS // NS
CHUNKS_PER_TILE = ROWS_PER_TILE // SUBS

DUMMY = N_NODES
ACC_ROWS = N_NODES + 8
STRIPE = 6256
LAST_STRIPE = N_NODES - (NS - 1) * STRIPE

_mesh = plsc.VectorSubcoreMesh(
    core_axis_name="c", subcore_axis_name="s", num_cores=NC, num_subcores=NS
)

_params = pltpu.CompilerParams(use_tc_tiling_on_sc=False,
                               needs_layout_passes=False)


def _layer_body(emb_in, src_hbm, dst_hbm, val_hbm, zeros, emb_out,
                src_v, dst_v, val_v, rows_v, acc_sh, sem):
    c = lax.axis_index("c")
    s = lax.axis_index("s")
    plane = c * N_NODES

    @pl.when(s < NS - 1)
    def _zero_full():
        pltpu.sync_copy(zeros, acc_sh.at[pl.ds(s * STRIPE, STRIPE)])

    @pl.when(s == NS - 1)
    def _zero_last():
        pltpu.sync_copy(zeros.at[pl.ds(0, LAST_STRIPE)],
                        acc_sh.at[pl.ds(s * STRIPE, LAST_STRIPE)])

    plsc.subcore_barrier()

    def chunk_body(i, carry):
        off = s * ROWS_PER_TILE + i * SUBS
        pltpu.async_copy(src_hbm.at[pl.ds(off, SUBS)], src_v, sem).wait()
        pltpu.async_copy(dst_hbm.at[pl.ds(off, SUBS)], dst_v, sem).wait()
        pltpu.async_copy(val_hbm.at[pl.ds(off, SUBS)], val_v, sem).wait()

        for j in range(SUBS):
            for k in range(SUB // 16):
                sv = src_v[j, pl.ds(k * 16, 16)]
                src_v[j, pl.ds(k * 16, 16)] = sv + plane

        for j in range(SUBS):
            pltpu.async_copy(emb_in.at[src_v.at[j]], rows_v, sem).wait()

            def scale_body(g, _):
                vv = val_v[j, pl.ds(g * 16, 16)]
                for i2 in range(16):
                    e = g * 16 + i2
                    vi = vv[i2]
                    rows_v[e, pl.ds(0, 16)] = rows_v[e, pl.ds(0, 16)] * vi
                return 0

            lax.fori_loop(0, SUB // 16, scale_body, 0)
            pltpu.async_copy(rows_v, acc_sh.at[dst_v.at[j]], sem,
                             add=True).wait()
        return carry

    lax.fori_loop(0, CHUNKS_PER_TILE, chunk_body, 0)

    plsc.subcore_barrier()

    @pl.when(s < NS - 1)
    def _wb_full():
        pltpu.sync_copy(acc_sh.at[pl.ds(s * STRIPE, STRIPE)],
                        emb_out.at[pl.ds(plane + s * STRIPE, STRIPE)])

    @pl.when(s == NS - 1)
    def _wb_last():
        pltpu.sync_copy(acc_sh.at[pl.ds(s * STRIPE, LAST_STRIPE)],
                        emb_out.at[pl.ds(plane + s * STRIPE, LAST_STRIPE)])


_layer = pl.kernel(
    _layer_body,
    out_type=jax.ShapeDtypeStruct((2 * N_NODES, HD), jnp.float32),
    mesh=_mesh,
    compiler_params=_params,
    scratch_types=[
        pltpu.VMEM((SUBS, SUB), jnp.int32),
        pltpu.VMEM((SUBS, SUB), jnp.int32),
        pltpu.VMEM((SUBS, SUB), jnp.float32),
        pltpu.VMEM((SUB, HD), jnp.float32),
        pltpu.VMEM_SHARED((ACC_ROWS, HD), jnp.float32),
        pltpu.SemaphoreType.DMA,
    ],
)

B_PER_W = BATCH // (NC * NS)
BROWS_PER_W = B_PER_W // SUB


def _final_body(emb0, emb1, emb2, emb3, users2d, pos2d,
                ue, pe, uf, pf, idx_v, idxb_v, acc_a, acc_b, acc32, sem):
    c = lax.axis_index("c")
    s = lax.axis_index("s")
    wid = s * NC + c
    row0 = wid * BROWS_PER_W
    base = wid * B_PER_W

    def interleave(scale):
        def body(i, _):
            acc32[i, pl.ds(0, 16)] = acc_a[i, pl.ds(0, 16)] * scale
            acc32[i, pl.ds(16, 16)] = acc_b[i, pl.ds(0, 16)] * scale
            return 0

        lax.fori_loop(0, B_PER_W, body, 0)

    def lookup(idx2d, offset, out_raw, out_final):
        pltpu.async_copy(idx2d.at[pl.ds(row0, BROWS_PER_W)], idx_v, sem).wait()
        for j in range(BROWS_PER_W):
            for k in range(SUB // 16):
                v = idx_v[j, pl.ds(k * 16, 16)] + offset
                idx_v[j, pl.ds(k * 16, 16)] = v
                idxb_v[j, pl.ds(k * 16, 16)] = v + N_NODES
        for j in range(BROWS_PER_W):
            pltpu.async_copy(emb0.at[idx_v.at[j]],
                             acc_a.at[pl.ds(j * SUB, SUB)], sem).wait()
            pltpu.async_copy(emb0.at[idxb_v.at[j]],
                             acc_b.at[pl.ds(j * SUB, SUB)], sem).wait()
        interleave(1.0)
        pltpu.sync_copy(acc32, out_raw.at[pl.ds(base, B_PER_W)])
        for emb in (emb1, emb2, emb3):
            for j in range(BROWS_PER_W):
                pltpu.async_copy(emb.at[idx_v.at[j]],
                                 acc_a.at[pl.ds(j * SUB, SUB)], sem,
                                 add=True).wait()
                pltpu.async_copy(emb.at[idxb_v.at[j]],
                                 acc_b.at[pl.ds(j * SUB, SUB)], sem,
                                 add=True).wait()
        interleave(0.25)
        pltpu.sync_copy(acc32, out_final.at[pl.ds(base, B_PER_W)])

    lookup(users2d, 0, ue, uf)
    lookup(pos2d, N_USERS, pe, pf)


_final = pl.kernel(
    _final_body,
    out_type=(
        jax.ShapeDtypeStruct((BATCH, D), jnp.float32),
        jax.ShapeDtypeStruct((BATCH, D), jnp.float32),
        jax.ShapeDtypeStruct((BATCH, D), jnp.float32),
        jax.ShapeDtypeStruct((BATCH, D), jnp.float32),
    ),
    mesh=_mesh,
    compiler_params=_params,
    scratch_types=[
        pltpu.VMEM((BROWS_PER_W, SUB), jnp.int32),
        pltpu.VMEM((BROWS_PER_W, SUB), jnp.int32),
        pltpu.VMEM((B_PER_W, HD), jnp.float32),
        pltpu.VMEM((B_PER_W, HD), jnp.float32),
        pltpu.VMEM((B_PER_W, D), jnp.float32),
        pltpu.SemaphoreType.DMA,
    ],
)


def kernel(user_table, item_table, edge_val, edge_src, edge_dst, users, pos_items):
    lo = jnp.concatenate([user_table[:, :HD], item_table[:, :HD]], axis=0)
    hi = jnp.concatenate([user_table[:, HD:], item_table[:, HD:]], axis=0)
    emb0 = jnp.concatenate([lo, hi], axis=0)

    pad = E_PAD - N_EDGES
    src_p = jnp.concatenate(
        [edge_src.astype(jnp.int32), jnp.zeros((pad,), jnp.int32)]
    ).reshape(ROWS, SUB)
    dst_p = jnp.concatenate(
        [edge_dst.astype(jnp.int32), jnp.full((pad,), DUMMY, jnp.int32)]
    ).reshape(ROWS, SUB)
    val_p = jnp.concatenate(
        [edge_val, jnp.zeros((pad,), jnp.float32)]
    ).reshape(ROWS, SUB)
    zeros = jnp.zeros((STRIPE, HD), jnp.float32)

    e1 = _layer(emb0, src_p, dst_p, val_p, zeros)
    e2 = _layer(e1, src_p, dst_p, val_p, zeros)
    e3 = _layer(e2, src_p, dst_p, val_p, zeros)

    users2d = users.astype(jnp.int32).reshape(BATCH // SUB, SUB)
    pos2d = pos_items.astype(jnp.int32).reshape(BATCH // SUB, SUB)
    return _final(emb0, e1, e2, e3, users2d, pos2d)


# speedup vs baseline: 7.9969x; 1.2225x over previous
import jax
import jax.numpy as jnp
from jax import lax
from jax.experimental import pallas as pl
from jax.experimental.pallas import tpu as pltpu
from jax.experimental.pallas import tpu_sc as plsc

N_USERS = 50000
N_ITEMS = 50000
N_NODES = N_USERS + N_ITEMS
N_EDGES = 1600000
D = 32
HD = 16
BATCH = 16384

NC = 2
NS = 16
SUB = 128
SUBS = 16
CHUNK = SUB * SUBS
E_PAD = ((N_EDGES + NS * CHUNK - 1) // (NS * CHUNK)) * (NS * CHUNK)
ROWS = E_PAD // SUB
ROWS_PER_TILE = ROWS // NS
CHUNKS_PER_TILE = ROWS_PER_TILE // SUBS

DUMMY = N_NODES
ACC_ROWS = N_NODES + 8
STRIPE = 6256
LAST_STRIPE = N_NODES - (NS - 1) * STRIPE

_mesh = plsc.VectorSubcoreMesh(
    core_axis_name="c", subcore_axis_name="s", num_cores=NC, num_subcores=NS
)

_params = pltpu.CompilerParams(use_tc_tiling_on_sc=False,
                               needs_layout_passes=False)


def _layer_body(emb_in, src_hbm, dst_hbm, val_hbm, zeros, emb_out,
                src_v, dst_v, val_v, rows_v, acc_sh, sem, gsem, ssem):
    c = lax.axis_index("c")
    s = lax.axis_index("s")
    plane = c * N_NODES

    @pl.when(s < NS - 1)
    def _zero_full():
        pltpu.sync_copy(zeros, acc_sh.at[pl.ds(s * STRIPE, STRIPE)])

    @pl.when(s == NS - 1)
    def _zero_last():
        pltpu.sync_copy(zeros.at[pl.ds(0, LAST_STRIPE)],
                        acc_sh.at[pl.ds(s * STRIPE, LAST_STRIPE)])

    plsc.subcore_barrier()

    def chunk_body(i, carry):
        off = s * ROWS_PER_TILE + i * SUBS
        pltpu.async_copy(src_hbm.at[pl.ds(off, SUBS)], src_v, sem).wait()
        pltpu.async_copy(dst_hbm.at[pl.ds(off, SUBS)], dst_v, sem).wait()
        pltpu.async_copy(val_hbm.at[pl.ds(off, SUBS)], val_v, sem).wait()

        for j in range(SUBS):
            for k in range(SUB // 16):
                sv = src_v[j, pl.ds(k * 16, 16)]
                src_v[j, pl.ds(k * 16, 16)] = sv + plane

        def gat(j):
            return pltpu.make_async_copy(emb_in.at[src_v.at[j]],
                                         rows_v.at[j & 1], gsem)

        def sct(j):
            return pltpu.make_async_copy(rows_v.at[j & 1],
                                         acc_sh.at[dst_v.at[j]], ssem)

        gat(0).start()
        for j in range(SUBS):
            q = j & 1
            if j >= 1:
                sct(j - 1).wait()
            gat(j).wait()
            if j + 1 < SUBS:
                gat(j + 1).start()

            def scale_body(g, _, j=j, q=q):
                vv = val_v[j, pl.ds(g * 16, 16)]
                for i2 in range(16):
                    e = g * 16 + i2
                    vi = vv[i2]
                    rows_v[q, e, pl.ds(0, 16)] = (
                        rows_v[q, e, pl.ds(0, 16)] * vi)
                return 0

            lax.fori_loop(0, SUB // 16, scale_body, 0)
            pltpu.async_copy(rows_v.at[q], acc_sh.at[dst_v.at[j]], ssem,
                             add=True)
        sct(SUBS - 1).wait()
        return carry

    lax.fori_loop(0, CHUNKS_PER_TILE, chunk_body, 0)

    plsc.subcore_barrier()

    @pl.when(s < NS - 1)
    def _wb_full():
        pltpu.sync_copy(acc_sh.at[pl.ds(s * STRIPE, STRIPE)],
                        emb_out.at[pl.ds(plane + s * STRIPE, STRIPE)])

    @pl.when(s == NS - 1)
    def _wb_last():
        pltpu.sync_copy(acc_sh.at[pl.ds(s * STRIPE, LAST_STRIPE)],
                        emb_out.at[pl.ds(plane + s * STRIPE, LAST_STRIPE)])


_layer = pl.kernel(
    _layer_body,
    out_type=jax.ShapeDtypeStruct((2 * N_NODES, HD), jnp.float32),
    mesh=_mesh,
    compiler_params=_params,
    scratch_types=[
        pltpu.VMEM((SUBS, SUB), jnp.int32),
        pltpu.VMEM((SUBS, SUB), jnp.int32),
        pltpu.VMEM((SUBS, SUB), jnp.float32),
        pltpu.VMEM((2, SUB, HD), jnp.float32),
        pltpu.VMEM_SHARED((ACC_ROWS, HD), jnp.float32),
        pltpu.SemaphoreType.DMA,
        pltpu.SemaphoreType.DMA,
        pltpu.SemaphoreType.DMA,
    ],
)

B_PER_W = BATCH // (NC * NS)
BROWS_PER_W = B_PER_W // SUB


def _final_body(emb0, emb1, emb2, emb3, users2d, pos2d,
                ue, pe, uf, pf, idx_v, idxb_v, acc_a, acc_b, acc32, sem):
    c = lax.axis_index("c")
    s = lax.axis_index("s")
    wid = s * NC + c
    row0 = wid * BROWS_PER_W
    base = wid * B_PER_W

    def interleave(scale):
        def body(i, _):
            acc32[i, pl.ds(0, 16)] = acc_a[i, pl.ds(0, 16)] * scale
            acc32[i, pl.ds(16, 16)] = acc_b[i, pl.ds(0, 16)] * scale
            return 0

        lax.fori_loop(0, B_PER_W, body, 0)

    def lookup(idx2d, offset, out_raw, out_final):
        pltpu.async_copy(idx2d.at[pl.ds(row0, BROWS_PER_W)], idx_v, sem).wait()
        for j in range(BROWS_PER_W):
            for k in range(SUB // 16):
                v = idx_v[j, pl.ds(k * 16, 16)] + offset
                idx_v[j, pl.ds(k * 16, 16)] = v
                idxb_v[j, pl.ds(k * 16, 16)] = v + N_NODES
        for j in range(BROWS_PER_W):
            pltpu.async_copy(emb0.at[idx_v.at[j]],
                             acc_a.at[pl.ds(j * SUB, SUB)], sem).wait()
            pltpu.async_copy(emb0.at[idxb_v.at[j]],
                             acc_b.at[pl.ds(j * SUB, SUB)], sem).wait()
        interleave(1.0)
        pltpu.sync_copy(acc32, out_raw.at[pl.ds(base, B_PER_W)])
        for emb in (emb1, emb2, emb3):
            for j in range(BROWS_PER_W):
                pltpu.async_copy(emb.at[idx_v.at[j]],
                                 acc_a.at[pl.ds(j * SUB, SUB)], sem,
                                 add=True).wait()
                pltpu.async_copy(emb.at[idxb_v.at[j]],
                                 acc_b.at[pl.ds(j * SUB, SUB)], sem,
                                 add=True).wait()
        interleave(0.25)
        pltpu.sync_copy(acc32, out_final.at[pl.ds(base, B_PER_W)])

    lookup(users2d, 0, ue, uf)
    lookup(pos2d, N_USERS, pe, pf)


_final = pl.kernel(
    _final_body,
    out_type=(
        jax.ShapeDtypeStruct((BATCH, D), jnp.float32),
        jax.ShapeDtypeStruct((BATCH, D), jnp.float32),
        jax.ShapeDtypeStruct((BATCH, D), jnp.float32),
        jax.ShapeDtypeStruct((BATCH, D), jnp.float32),
    ),
    mesh=_mesh,
    compiler_params=_params,
    scratch_types=[
        pltpu.VMEM((BROWS_PER_W, SUB), jnp.int32),
        pltpu.VMEM((BROWS_PER_W, SUB), jnp.int32),
        pltpu.VMEM((B_PER_W, HD), jnp.float32),
        pltpu.VMEM((B_PER_W, HD), jnp.float32),
        pltpu.VMEM((B_PER_W, D), jnp.float32),
        pltpu.SemaphoreType.DMA,
    ],
)


def kernel(user_table, item_table, edge_val, edge_src, edge_dst, users, pos_items):
    lo = jnp.concatenate([user_table[:, :HD], item_table[:, :HD]], axis=0)
    hi = jnp.concatenate([user_table[:, HD:], item_table[:, HD:]], axis=0)
    emb0 = jnp.concatenate([lo, hi], axis=0)

    pad = E_PAD - N_EDGES
    src_p = jnp.concatenate(
        [edge_src.astype(jnp.int32), jnp.zeros((pad,), jnp.int32)]
    ).reshape(ROWS, SUB)
    dst_p = jnp.concatenate(
        [edge_dst.astype(jnp.int32), jnp.full((pad,), DUMMY, jnp.int32)]
    ).reshape(ROWS, SUB)
    val_p = jnp.concatenate(
        [edge_val, jnp.zeros((pad,), jnp.float32)]
    ).reshape(ROWS, SUB)
    zeros = jnp.zeros((STRIPE, HD), jnp.float32)

    e1 = _layer(emb0, src_p, dst_p, val_p, zeros)
    e2 = _layer(e1, src_p, dst_p, val_p, zeros)
    e3 = _layer(e2, src_p, dst_p, val_p, zeros)

    users2d = users.astype(jnp.int32).reshape(BATCH // SUB, SUB)
    pos2d = pos_items.astype(jnp.int32).reshape(BATCH // SUB, SUB)
    return _final(emb0, e1, e2, e3, users2d, pos2d)


# speedup vs baseline: 12.0557x; 1.5075x over previous
import jax
import jax.numpy as jnp
from jax import lax
from jax.experimental import pallas as pl
from jax.experimental.pallas import tpu as pltpu
from jax.experimental.pallas import tpu_sc as plsc

N_USERS = 50000
N_ITEMS = 50000
N_NODES = N_USERS + N_ITEMS
N_EDGES = 1600000
D = 32
HD = 16
BATCH = 16384

NC = 2
NS = 16
SUB = 128
SUBS = 16
CHUNK = SUB * SUBS
E_PAD = ((N_EDGES + NS * CHUNK - 1) // (NS * CHUNK)) * (NS * CHUNK)
ROWS = E_PAD // SUB
ROWS_PER_TILE = ROWS // NS
CHUNKS_PER_TILE = ROWS_PER_TILE // SUBS

DUMMY = N_NODES
ACC_ROWS = N_NODES + 8
STRIPE = 6256
LAST_STRIPE = N_NODES - (NS - 1) * STRIPE

_mesh = plsc.VectorSubcoreMesh(
    core_axis_name="c", subcore_axis_name="s", num_cores=NC, num_subcores=NS
)

_params = pltpu.CompilerParams(use_tc_tiling_on_sc=False,
                               needs_layout_passes=False)


def _layer_body(emb_in, src_hbm, dst_hbm, val_hbm, zeros, emb_out,
                src_v, dst_v, val_v, rows_v, acc_sh, sem,
                g0, g1, g2, g3, s0, s1, s2, s3):
    gsems = (g0, g1, g2, g3)
    ssems = (s0, s1, s2, s3)
    c = lax.axis_index("c")
    s = lax.axis_index("s")
    plane = c * N_NODES

    @pl.when(s < NS - 1)
    def _zero_full():
        pltpu.sync_copy(zeros, acc_sh.at[pl.ds(s * STRIPE, STRIPE)])

    @pl.when(s == NS - 1)
    def _zero_last():
        pltpu.sync_copy(zeros.at[pl.ds(0, LAST_STRIPE)],
                        acc_sh.at[pl.ds(s * STRIPE, LAST_STRIPE)])

    plsc.subcore_barrier()

    def chunk_body(i, carry):
        off = s * ROWS_PER_TILE + i * SUBS
        pltpu.async_copy(src_hbm.at[pl.ds(off, SUBS)], src_v, sem).wait()
        pltpu.async_copy(dst_hbm.at[pl.ds(off, SUBS)], dst_v, sem).wait()
        pltpu.async_copy(val_hbm.at[pl.ds(off, SUBS)], val_v, sem).wait()

        for j in range(SUBS):
            for k in range(SUB // 16):
                sv = src_v[j, pl.ds(k * 16, 16)]
                src_v[j, pl.ds(k * 16, 16)] = sv + plane

        def gat(j):
            return pltpu.make_async_copy(emb_in.at[src_v.at[j]],
                                         rows_v.at[j % 4], gsems[j % 4])

        def sct(j):
            return pltpu.make_async_copy(rows_v.at[j % 4],
                                         acc_sh.at[dst_v.at[j]], ssems[j % 4])

        gat(0).start()
        gat(1).start()
        for j in range(SUBS):
            q = j % 4
            if j >= 2:
                sct(j - 2).wait()
            gat(j).wait()
            if j + 2 < SUBS:
                gat(j + 2).start()

            def scale_body(g, _, j=j, q=q):
                vv = val_v[j, pl.ds(g * 16, 16)]
                for i2 in range(16):
                    e = g * 16 + i2
                    vi = vv[i2]
                    rows_v[q, e, pl.ds(0, 16)] = (
                        rows_v[q, e, pl.ds(0, 16)] * vi)
                return 0

            lax.fori_loop(0, SUB // 16, scale_body, 0)
            pltpu.async_copy(rows_v.at[q], acc_sh.at[dst_v.at[j]],
                             ssems[q], add=True)
        sct(SUBS - 2).wait()
        sct(SUBS - 1).wait()
        return carry

    lax.fori_loop(0, CHUNKS_PER_TILE, chunk_body, 0)

    plsc.subcore_barrier()

    @pl.when(s < NS - 1)
    def _wb_full():
        pltpu.sync_copy(acc_sh.at[pl.ds(s * STRIPE, STRIPE)],
                        emb_out.at[pl.ds(plane + s * STRIPE, STRIPE)])

    @pl.when(s == NS - 1)
    def _wb_last():
        pltpu.sync_copy(acc_sh.at[pl.ds(s * STRIPE, LAST_STRIPE)],
                        emb_out.at[pl.ds(plane + s * STRIPE, LAST_STRIPE)])


_layer = pl.kernel(
    _layer_body,
    out_type=jax.ShapeDtypeStruct((2 * N_NODES, HD), jnp.float32),
    mesh=_mesh,
    compiler_params=_params,
    scratch_types=[
        pltpu.VMEM((SUBS, SUB), jnp.int32),
        pltpu.VMEM((SUBS, SUB), jnp.int32),
        pltpu.VMEM((SUBS, SUB), jnp.float32),
        pltpu.VMEM((4, SUB, HD), jnp.float32),
        pltpu.VMEM_SHARED((ACC_ROWS, HD), jnp.float32),
        pltpu.SemaphoreType.DMA,
        pltpu.SemaphoreType.DMA,
        pltpu.SemaphoreType.DMA,
        pltpu.SemaphoreType.DMA,
        pltpu.SemaphoreType.DMA,
        pltpu.SemaphoreType.DMA,
        pltpu.SemaphoreType.DMA,
        pltpu.SemaphoreType.DMA,
        pltpu.SemaphoreType.DMA,
    ],
)

B_PER_W = BATCH // (NC * NS)
BROWS_PER_W = B_PER_W // SUB


def _final_body(emb0, emb1, emb2, emb3, users2d, pos2d,
                ue, pe, uf, pf, idx_v, idxb_v, acc_a, acc_b, acc32, sem):
    c = lax.axis_index("c")
    s = lax.axis_index("s")
    wid = s * NC + c
    row0 = wid * BROWS_PER_W
    base = wid * B_PER_W

    def interleave(scale):
        def body(i, _):
            acc32[i, pl.ds(0, 16)] = acc_a[i, pl.ds(0, 16)] * scale
            acc32[i, pl.ds(16, 16)] = acc_b[i, pl.ds(0, 16)] * scale
            return 0

        lax.fori_loop(0, B_PER_W, body, 0)

    def lookup(idx2d, offset, out_raw, out_final):
        pltpu.async_copy(idx2d.at[pl.ds(row0, BROWS_PER_W)], idx_v, sem).wait()
        for j in range(BROWS_PER_W):
            for k in range(SUB // 16):
                v = idx_v[j, pl.ds(k * 16, 16)] + offset
                idx_v[j, pl.ds(k * 16, 16)] = v
                idxb_v[j, pl.ds(k * 16, 16)] = v + N_NODES
        for j in range(BROWS_PER_W):
            pltpu.async_copy(emb0.at[idx_v.at[j]],
                             acc_a.at[pl.ds(j * SUB, SUB)], sem).wait()
            pltpu.async_copy(emb0.at[idxb_v.at[j]],
                             acc_b.at[pl.ds(j * SUB, SUB)], sem).wait()
        interleave(1.0)
        pltpu.sync_copy(acc32, out_raw.at[pl.ds(base, B_PER_W)])
        for emb in (emb1, emb2, emb3):
            for j in range(BROWS_PER_W):
                pltpu.async_copy(emb.at[idx_v.at[j]],
                                 acc_a.at[pl.ds(j * SUB, SUB)], sem,
                                 add=True).wait()
                pltpu.async_copy(emb.at[idxb_v.at[j]],
                                 acc_b.at[pl.ds(j * SUB, SUB)], sem,
                                 add=True).wait()
        interleave(0.25)
        pltpu.sync_copy(acc32, out_final.at[pl.ds(base, B_PER_W)])

    lookup(users2d, 0, ue, uf)
    lookup(pos2d, N_USERS, pe, pf)


_final = pl.kernel(
    _final_body,
    out_type=(
        jax.ShapeDtypeStruct((BATCH, D), jnp.float32),
        jax.ShapeDtypeStruct((BATCH, D), jnp.float32),
        jax.ShapeDtypeStruct((BATCH, D), jnp.float32),
        jax.ShapeDtypeStruct((BATCH, D), jnp.float32),
    ),
    mesh=_mesh,
    compiler_params=_params,
    scratch_types=[
        pltpu.VMEM((BROWS_PER_W, SUB), jnp.int32),
        pltpu.VMEM((BROWS_PER_W, SUB), jnp.int32),
        pltpu.VMEM((B_PER_W, HD), jnp.float32),
        pltpu.VMEM((B_PER_W, HD), jnp.float32),
        pltpu.VMEM((B_PER_W, D), jnp.float32),
        pltpu.SemaphoreType.DMA,
    ],
)


def kernel(user_table, item_table, edge_val, edge_src, edge_dst, users, pos_items):
    lo = jnp.concatenate([user_table[:, :HD], item_table[:, :HD]], axis=0)
    hi = jnp.concatenate([user_table[:, HD:], item_table[:, HD:]], axis=0)
    emb0 = jnp.concatenate([lo, hi], axis=0)

    pad = E_PAD - N_EDGES
    src_p = jnp.concatenate(
        [edge_src.astype(jnp.int32), jnp.zeros((pad,), jnp.int32)]
    ).reshape(ROWS, SUB)
    dst_p = jnp.concatenate(
        [edge_dst.astype(jnp.int32), jnp.full((pad,), DUMMY, jnp.int32)]
    ).reshape(ROWS, SUB)
    val_p = jnp.concatenate(
        [edge_val, jnp.zeros((pad,), jnp.float32)]
    ).reshape(ROWS, SUB)
    zeros = jnp.zeros((STRIPE, HD), jnp.float32)

    e1 = _layer(emb0, src_p, dst_p, val_p, zeros)
    e2 = _layer(e1, src_p, dst_p, val_p, zeros)
    e3 = _layer(e2, src_p, dst_p, val_p, zeros)

    users2d = users.astype(jnp.int32).reshape(BATCH // SUB, SUB)
    pos2d = pos_items.astype(jnp.int32).reshape(BATCH // SUB, SUB)
    return _final(emb0, e1, e2, e3, users2d, pos2d)


# speedup vs baseline: 15.2819x; 1.2676x over previous
import jax
import jax.numpy as jnp
from jax import lax
from jax.experimental import pallas as pl
from jax.experimental.pallas import tpu as pltpu
from jax.experimental.pallas import tpu_sc as plsc

N_USERS = 50000
N_ITEMS = 50000
N_NODES = N_USERS + N_ITEMS
N_EDGES = 1600000
D = 32
HD = 16
BATCH = 16384

NC = 2
NS = 16
SUB = 128
SUBS = 16
CHUNK = SUB * SUBS
E_PAD = ((N_EDGES + NS * CHUNK - 1) // (NS * CHUNK)) * (NS * CHUNK)
ROWS = E_PAD // SUB
ROWS_PER_TILE = ROWS // NS
CHUNKS_PER_TILE = ROWS_PER_TILE // SUBS

DUMMY = N_NODES
ACC_ROWS = N_NODES + 8
STRIPE = 6256
LAST_STRIPE = N_NODES - (NS - 1) * STRIPE

_mesh = plsc.VectorSubcoreMesh(
    core_axis_name="c", subcore_axis_name="s", num_cores=NC, num_subcores=NS
)

_params = pltpu.CompilerParams(use_tc_tiling_on_sc=False,
                               needs_layout_passes=False)


def _layer_body(emb_in, src_hbm, dst_hbm, val_hbm, zeros, emb_out,
                src_v, dst_v, val_v, rows_v, acc_sh, sem,
                g0, g1, g2, g3, g4, g5, g6, g7,
                s0, s1, s2, s3, s4, s5, s6, s7):
    gsems = (g0, g1, g2, g3, g4, g5, g6, g7)
    ssems = (s0, s1, s2, s3, s4, s5, s6, s7)
    c = lax.axis_index("c")
    s = lax.axis_index("s")
    plane = c * N_NODES

    @pl.when(s < NS - 1)
    def _zero_full():
        pltpu.sync_copy(zeros, acc_sh.at[pl.ds(s * STRIPE, STRIPE)])

    @pl.when(s == NS - 1)
    def _zero_last():
        pltpu.sync_copy(zeros.at[pl.ds(0, LAST_STRIPE)],
                        acc_sh.at[pl.ds(s * STRIPE, LAST_STRIPE)])

    plsc.subcore_barrier()

    def chunk_body(i, carry):
        off = s * ROWS_PER_TILE + i * SUBS
        pltpu.async_copy(src_hbm.at[pl.ds(off, SUBS)], src_v, sem).wait()
        pltpu.async_copy(dst_hbm.at[pl.ds(off, SUBS)], dst_v, sem).wait()
        pltpu.async_copy(val_hbm.at[pl.ds(off, SUBS)], val_v, sem).wait()

        for j in range(SUBS):
            for k in range(SUB // 16):
                sv = src_v[j, pl.ds(k * 16, 16)]
                src_v[j, pl.ds(k * 16, 16)] = sv + plane

        def gat(j):
            return pltpu.make_async_copy(emb_in.at[src_v.at[j]],
                                         rows_v.at[j % 8], gsems[j % 8])

        def sct(j):
            return pltpu.make_async_copy(rows_v.at[j % 8],
                                         acc_sh.at[dst_v.at[j]], ssems[j % 8])

        for jp in range(4):
            gat(jp).start()
        for j in range(SUBS):
            q = j % 8
            if j >= 4:
                sct(j - 4).wait()
            gat(j).wait()
            if j + 4 < SUBS:
                gat(j + 4).start()

            def scale_body(g, _, j=j, q=q):
                vv = val_v[j, pl.ds(g * 16, 16)]
                for i2 in range(16):
                    e = g * 16 + i2
                    vi = vv[i2]
                    rows_v[q, e, pl.ds(0, 16)] = (
                        rows_v[q, e, pl.ds(0, 16)] * vi)
                return 0

            lax.fori_loop(0, SUB // 16, scale_body, 0)
            pltpu.async_copy(rows_v.at[q], acc_sh.at[dst_v.at[j]],
                             ssems[q], add=True)
        for jt in range(SUBS - 4, SUBS):
            sct(jt).wait()
        return carry

    lax.fori_loop(0, CHUNKS_PER_TILE, chunk_body, 0)

    plsc.subcore_barrier()

    @pl.when(s < NS - 1)
    def _wb_full():
        pltpu.sync_copy(acc_sh.at[pl.ds(s * STRIPE, STRIPE)],
                        emb_out.at[pl.ds(plane + s * STRIPE, STRIPE)])

    @pl.when(s == NS - 1)
    def _wb_last():
        pltpu.sync_copy(acc_sh.at[pl.ds(s * STRIPE, LAST_STRIPE)],
                        emb_out.at[pl.ds(plane + s * STRIPE, LAST_STRIPE)])


_layer = pl.kernel(
    _layer_body,
    out_type=jax.ShapeDtypeStruct((2 * N_NODES, HD), jnp.float32),
    mesh=_mesh,
    compiler_params=_params,
    scratch_types=[
        pltpu.VMEM((SUBS, SUB), jnp.int32),
        pltpu.VMEM((SUBS, SUB), jnp.int32),
        pltpu.VMEM((SUBS, SUB), jnp.float32),
        pltpu.VMEM((8, SUB, HD), jnp.float32),
        pltpu.VMEM_SHARED((ACC_ROWS, HD), jnp.float32),
        pltpu.SemaphoreType.DMA,
    ] + [pltpu.SemaphoreType.DMA] * 16,
)

B_PER_W = BATCH // (NC * NS)
BROWS_PER_W = B_PER_W // SUB


def _final_body(emb0, emb1, emb2, emb3, users2d, pos2d,
                ue, pe, uf, pf, idx_v, idxb_v, acc_a, acc_b, acc32, sem):
    c = lax.axis_index("c")
    s = lax.axis_index("s")
    wid = s * NC + c
    row0 = wid * BROWS_PER_W
    base = wid * B_PER_W

    def interleave(scale):
        def body(i, _):
            acc32[i, pl.ds(0, 16)] = acc_a[i, pl.ds(0, 16)] * scale
            acc32[i, pl.ds(16, 16)] = acc_b[i, pl.ds(0, 16)] * scale
            return 0

        lax.fori_loop(0, B_PER_W, body, 0)

    def lookup(idx2d, offset, out_raw, out_final):
        pltpu.async_copy(idx2d.at[pl.ds(row0, BROWS_PER_W)], idx_v, sem).wait()
        for j in range(BROWS_PER_W):
            for k in range(SUB // 16):
                v = idx_v[j, pl.ds(k * 16, 16)] + offset
                idx_v[j, pl.ds(k * 16, 16)] = v
                idxb_v[j, pl.ds(k * 16, 16)] = v + N_NODES
        for j in range(BROWS_PER_W):
            pltpu.async_copy(emb0.at[idx_v.at[j]],
                             acc_a.at[pl.ds(j * SUB, SUB)], sem).wait()
            pltpu.async_copy(emb0.at[idxb_v.at[j]],
                             acc_b.at[pl.ds(j * SUB, SUB)], sem).wait()
        interleave(1.0)
        pltpu.sync_copy(acc32, out_raw.at[pl.ds(base, B_PER_W)])
        for emb in (emb1, emb2, emb3):
            for j in range(BROWS_PER_W):
                pltpu.async_copy(emb.at[idx_v.at[j]],
                                 acc_a.at[pl.ds(j * SUB, SUB)], sem,
                                 add=True).wait()
                pltpu.async_copy(emb.at[idxb_v.at[j]],
                                 acc_b.at[pl.ds(j * SUB, SUB)], sem,
                                 add=True).wait()
        interleave(0.25)
        pltpu.sync_copy(acc32, out_final.at[pl.ds(base, B_PER_W)])

    lookup(users2d, 0, ue, uf)
    lookup(pos2d, N_USERS, pe, pf)


_final = pl.kernel(
    _final_body,
    out_type=(
        jax.ShapeDtypeStruct((BATCH, D), jnp.float32),
        jax.ShapeDtypeStruct((BATCH, D), jnp.float32),
        jax.ShapeDtypeStruct((BATCH, D), jnp.float32),
        jax.ShapeDtypeStruct((BATCH, D), jnp.float32),
    ),
    mesh=_mesh,
    compiler_params=_params,
    scratch_types=[
        pltpu.VMEM((BROWS_PER_W, SUB), jnp.int32),
        pltpu.VMEM((BROWS_PER_W, SUB), jnp.int32),
        pltpu.VMEM((B_PER_W, HD), jnp.float32),
        pltpu.VMEM((B_PER_W, HD), jnp.float32),
        pltpu.VMEM((B_PER_W, D), jnp.float32),
        pltpu.SemaphoreType.DMA,
    ],
)


def kernel(user_table, item_table, edge_val, edge_src, edge_dst, users, pos_items):
    lo = jnp.concatenate([user_table[:, :HD], item_table[:, :HD]], axis=0)
    hi = jnp.concatenate([user_table[:, HD:], item_table[:, HD:]], axis=0)
    emb0 = jnp.concatenate([lo, hi], axis=0)

    pad = E_PAD - N_EDGES
    src_p = jnp.concatenate(
        [edge_src.astype(jnp.int32), jnp.zeros((pad,), jnp.int32)]
    ).reshape(ROWS, SUB)
    dst_p = jnp.concatenate(
        [edge_dst.astype(jnp.int32), jnp.full((pad,), DUMMY, jnp.int32)]
    ).reshape(ROWS, SUB)
    val_p = jnp.concatenate(
        [edge_val, jnp.zeros((pad,), jnp.float32)]
    ).reshape(ROWS, SUB)
    zeros = jnp.zeros((STRIPE, HD), jnp.float32)

    e1 = _layer(emb0, src_p, dst_p, val_p, zeros)
    e2 = _layer(e1, src_p, dst_p, val_p, zeros)
    e3 = _layer(e2, src_p, dst_p, val_p, zeros)

    users2d = users.astype(jnp.int32).reshape(BATCH // SUB, SUB)
    pos2d = pos_items.astype(jnp.int32).reshape(BATCH // SUB, SUB)
    return _final(emb0, e1, e2, e3, users2d, pos2d)
